# Initial kernel scaffold; baseline (speedup 1.0000x reference)
import jax
import jax.numpy as jnp
from jax.experimental import pallas as pl


def kernel(encoded_graphs, banned_acts, W1, b1, W2, b2):
    raise NotImplementedError("write your pallas kernel here")


# speedup vs baseline: 7.2467x; 7.2467x over previous
import functools

import jax
import jax.numpy as jnp
import numpy as np
from jax import lax
from jax.experimental import pallas as pl
from jax.experimental.pallas import tpu as pltpu
from jax.experimental.pallas import tpu_sc as plsc

B = 16384
IN_DIM = 169
IN_PAD = 256
HIDDEN = 2048
ACTIONS = 1024
N_BANNED = 4096
MIN_VAL = float(np.finfo(np.float32).min)

TILE = 512
GRID = B // TILE

NUM_CORES = 2
NUM_SUBCORES = 16
NW = NUM_CORES * NUM_SUBCORES
ROWS_PER_W = B // NW
CHUNK_ROWS = 64
CHUNK = CHUNK_ROWS * ACTIONS
CHUNKS_PER_W = ROWS_PER_W // CHUNK_ROWS
LANES = 16


@functools.lru_cache(maxsize=1)
def _make_mask_kernel():
    mesh = plsc.VectorSubcoreMesh(core_axis_name="c", subcore_axis_name="s")

    @functools.partial(
        pl.kernel,
        mesh=mesh,
        out_type=jax.ShapeDtypeStruct((B * ACTIONS,), jnp.float32),
        scratch_types=[
            pltpu.VMEM((N_BANNED,), jnp.int32),
            pltpu.VMEM((CHUNK,), jnp.float32),
        ],
        compiler_params=pltpu.CompilerParams(needs_layout_passes=False),
    )
    def mask_kernel(banned_hbm, mask_hbm, banned_v, buf_v):
        wid = lax.axis_index("s") * NUM_CORES + lax.axis_index("c")
        pltpu.sync_copy(banned_hbm, banned_v)
        zeros16 = jnp.zeros((LANES,), jnp.float32)
        minval16 = jnp.full((LANES,), MIN_VAL, jnp.float32)

        def chunk_body(ci, carry):
            base = (wid * CHUNKS_PER_W + ci) * CHUNK

            def zbody(i, c):
                buf_v[pl.ds(i * LANES, LANES)] = zeros16
                return c

            lax.fori_loop(0, CHUNK // LANES, zbody, 0)

            def sbody(j, c):
                idx = banned_v[pl.ds(j * LANES, LANES)]
                local = idx - base
                ok = (local >= 0) & (local < CHUNK)
                safe = jnp.clip(local, 0, CHUNK - 1)
                plsc.store_scatter(buf_v, [safe], minval16, mask=ok)
                return c

            lax.fori_loop(0, N_BANNED // LANES, sbody, 0)
            pltpu.sync_copy(buf_v, mask_hbm.at[pl.ds(base, CHUNK)])
            return carry

        lax.fori_loop(0, CHUNKS_PER_W, chunk_body, 0)

    return mask_kernel


def _tc_body(x_ref, w1_ref, b1_ref, w2_ref, b2_ref, m_ref,
             raw_ref, idx_ref, val_ref):
    h = jnp.maximum(
        jnp.dot(x_ref[...], w1_ref[...], preferred_element_type=jnp.float32)
        + b1_ref[...], 0.0)
    raw = (jnp.dot(h, w2_ref[...], preferred_element_type=jnp.float32)
           + b2_ref[...])
    raw_ref[...] = raw
    q = jnp.where(m_ref[...] < 0.0, MIN_VAL, raw)
    vmax = jnp.max(q, axis=1, keepdims=True)
    col = lax.broadcasted_iota(jnp.int32, q.shape, 1)
    idx_ref[...] = jnp.min(jnp.where(q == vmax, col, jnp.int32(ACTIONS)),
                           axis=1, keepdims=True)
    val_ref[...] = vmax


_tc_call = pl.pallas_call(
    _tc_body,
    grid=(GRID,),
    in_specs=[
        pl.BlockSpec((TILE, IN_PAD), lambda i: (i, 0)),
        pl.BlockSpec((IN_PAD, HIDDEN), lambda i: (0, 0)),
        pl.BlockSpec((1, HIDDEN), lambda i: (0, 0)),
        pl.BlockSpec((HIDDEN, ACTIONS), lambda i: (0, 0)),
        pl.BlockSpec((1, ACTIONS), lambda i: (0, 0)),
        pl.BlockSpec((TILE, ACTIONS), lambda i: (i, 0)),
    ],
    out_specs=[
        pl.BlockSpec((TILE, ACTIONS), lambda i: (i, 0)),
        pl.BlockSpec((TILE, 1), lambda i: (i, 0)),
        pl.BlockSpec((TILE, 1), lambda i: (i, 0)),
    ],
    out_shape=[
        jax.ShapeDtypeStruct((B, ACTIONS), jnp.float32),
        jax.ShapeDtypeStruct((B, 1), jnp.int32),
        jax.ShapeDtypeStruct((B, 1), jnp.float32),
    ],
    compiler_params=pltpu.CompilerParams(
        dimension_semantics=("arbitrary",)),
)


def kernel(encoded_graphs, banned_acts, W1, b1, W2, b2):
    xp = jnp.pad(encoded_graphs, ((0, 0), (0, IN_PAD - IN_DIM)))
    W1p = jnp.pad(W1, ((0, IN_PAD - IN_DIM), (0, 0)))
    mask = _make_mask_kernel()(banned_acts.astype(jnp.int32))
    mask2d = mask.reshape(B, ACTIONS)
    raw, idx, val = _tc_call(xp, W1p, b1.reshape(1, HIDDEN), W2,
                             b2.reshape(1, ACTIONS), mask2d)
    return (idx, val, raw)


# speedup vs baseline: 10.4291x; 1.4391x over previous
import functools

import jax
import jax.numpy as jnp
import numpy as np
from jax import lax
from jax.experimental import pallas as pl
from jax.experimental.pallas import tpu as pltpu
from jax.experimental.pallas import tpu_sc as plsc

B = 16384
IN_DIM = 169
IN_PAD = 256
HIDDEN = 2048
ACTIONS = 1024
N_BANNED = 4096
MIN_VAL = float(np.finfo(np.float32).min)

TILE = 512
GRID = B // TILE

NUM_CORES = 2
NUM_SUBCORES = 16
NW = NUM_CORES * NUM_SUBCORES
ROWS_PER_W = B // NW
CHUNK_ROWS = 32
CHUNK = CHUNK_ROWS * ACTIONS
CHUNKS_PER_W = ROWS_PER_W // CHUNK_ROWS
LANES = 16


@functools.lru_cache(maxsize=1)
def _make_mask_kernel():
    mesh = plsc.VectorSubcoreMesh(core_axis_name="c", subcore_axis_name="s")

    zu = 16
    su = 4

    @functools.partial(
        pl.kernel,
        mesh=mesh,
        out_type=jax.ShapeDtypeStruct((B * ACTIONS,), jnp.float32),
        scratch_types=[
            pltpu.VMEM((N_BANNED,), jnp.int32),
            pltpu.VMEM((CHUNK,), jnp.float32),
            pltpu.VMEM((CHUNK,), jnp.float32),
            pltpu.SemaphoreType.DMA,
            pltpu.SemaphoreType.DMA,
        ],
        compiler_params=pltpu.CompilerParams(needs_layout_passes=False),
    )
    def mask_kernel(banned_hbm, mask_hbm, banned_v, buf_a, buf_b, sem_a, sem_b):
        wid = lax.axis_index("s") * NUM_CORES + lax.axis_index("c")
        pltpu.sync_copy(banned_hbm, banned_v)
        zeros16 = jnp.zeros((LANES,), jnp.float32)
        minval16 = jnp.full((LANES,), MIN_VAL, jnp.float32)
        bufs = (buf_a, buf_b)
        sems = (sem_a, sem_b)
        pending = [None, None]

        for ci in range(CHUNKS_PER_W):
            buf = bufs[ci % 2]
            sem = sems[ci % 2]
            if pending[ci % 2] is not None:
                pending[ci % 2].wait()
            base = (wid * CHUNKS_PER_W + ci) * CHUNK

            def zbody(i, c):
                for u in range(zu):
                    buf[pl.ds((i * zu + u) * LANES, LANES)] = zeros16
                return c

            lax.fori_loop(0, CHUNK // (LANES * zu), zbody, 0)

            def sbody(j, c):
                for u in range(su):
                    idx = banned_v[pl.ds((j * su + u) * LANES, LANES)]
                    local = idx - base
                    ok = (local >= 0) & (local < CHUNK)
                    safe = jnp.clip(local, 0, CHUNK - 1)
                    plsc.store_scatter(buf, [safe], minval16, mask=ok)
                return c

            lax.fori_loop(0, N_BANNED // (LANES * su), sbody, 0)
            pending[ci % 2] = pltpu.async_copy(
                buf, mask_hbm.at[pl.ds(base, CHUNK)], sem)
        for cp in pending:
            if cp is not None:
                cp.wait()

    return mask_kernel


def _tc_body(x_ref, w1_ref, b1_ref, w2_ref, b2_ref, m_ref,
             raw_ref, idx_ref, val_ref):
    h = jnp.maximum(
        jnp.dot(x_ref[...], w1_ref[...], preferred_element_type=jnp.float32)
        + b1_ref[...], 0.0)
    raw = (jnp.dot(h, w2_ref[...], preferred_element_type=jnp.float32)
           + b2_ref[...])
    raw_ref[...] = raw
    q = jnp.where(m_ref[...] < 0.0, MIN_VAL, raw)
    vmax = jnp.max(q, axis=1, keepdims=True)
    col = lax.broadcasted_iota(jnp.int32, q.shape, 1)
    idx_ref[...] = jnp.min(jnp.where(q == vmax, col, jnp.int32(ACTIONS)),
                           axis=1, keepdims=True)
    val_ref[...] = vmax


_tc_call = pl.pallas_call(
    _tc_body,
    grid=(GRID,),
    in_specs=[
        pl.BlockSpec((TILE, IN_PAD), lambda i: (i, 0)),
        pl.BlockSpec((IN_PAD, HIDDEN), lambda i: (0, 0)),
        pl.BlockSpec((1, HIDDEN), lambda i: (0, 0)),
        pl.BlockSpec((HIDDEN, ACTIONS), lambda i: (0, 0)),
        pl.BlockSpec((1, ACTIONS), lambda i: (0, 0)),
        pl.BlockSpec((TILE, ACTIONS), lambda i: (i, 0)),
    ],
    out_specs=[
        pl.BlockSpec((TILE, ACTIONS), lambda i: (i, 0)),
        pl.BlockSpec((TILE, 1), lambda i: (i, 0)),
        pl.BlockSpec((TILE, 1), lambda i: (i, 0)),
    ],
    out_shape=[
        jax.ShapeDtypeStruct((B, ACTIONS), jnp.float32),
        jax.ShapeDtypeStruct((B, 1), jnp.int32),
        jax.ShapeDtypeStruct((B, 1), jnp.float32),
    ],
    compiler_params=pltpu.CompilerParams(
        dimension_semantics=("arbitrary",)),
)


def kernel(encoded_graphs, banned_acts, W1, b1, W2, b2):
    xp = jnp.pad(encoded_graphs, ((0, 0), (0, IN_PAD - IN_DIM)))
    W1p = jnp.pad(W1, ((0, IN_PAD - IN_DIM), (0, 0)))
    mask = _make_mask_kernel()(banned_acts.astype(jnp.int32))
    mask2d = mask.reshape(B, ACTIONS)
    raw, idx, val = _tc_call(xp, W1p, b1.reshape(1, HIDDEN), W2,
                             b2.reshape(1, ACTIONS), mask2d)
    return (idx, val, raw)


# speedup vs baseline: 14.2680x; 1.3681x over previous
import functools

import jax
import jax.numpy as jnp
import numpy as np
from jax import lax
from jax.experimental import pallas as pl
from jax.experimental.pallas import tpu as pltpu
from jax.experimental.pallas import tpu_sc as plsc

B = 16384
IN_DIM = 169
IN_PAD = 256
HIDDEN = 2048
ACTIONS = 1024
N_BANNED = 4096
MIN_VAL = float(np.finfo(np.float32).min)

TILE = 512
GRID = B // TILE

NUM_CORES = 2
NUM_SUBCORES = 16
NW = NUM_CORES * NUM_SUBCORES
ROWS_PER_W = B // NW
CHUNK_ROWS = 32
CHUNK = CHUNK_ROWS * ACTIONS
CHUNKS_PER_W = ROWS_PER_W // CHUNK_ROWS
LANES = 16


@functools.lru_cache(maxsize=1)
def _make_mask_kernel():
    mesh = plsc.VectorSubcoreMesh(core_axis_name="c", subcore_axis_name="s")

    zu = 16
    su = 4

    @functools.partial(
        pl.kernel,
        mesh=mesh,
        out_type=jax.ShapeDtypeStruct((B * ACTIONS,), jnp.float32),
        scratch_types=[
            pltpu.VMEM((N_BANNED,), jnp.int32),
            pltpu.VMEM((CHUNK,), jnp.float32),
            pltpu.VMEM((CHUNK,), jnp.float32),
            pltpu.SemaphoreType.DMA,
            pltpu.SemaphoreType.DMA,
        ],
        compiler_params=pltpu.CompilerParams(needs_layout_passes=False),
    )
    def mask_kernel(banned_hbm, mask_hbm, banned_v, buf_a, buf_b, sem_a, sem_b):
        wid = lax.axis_index("s") * NUM_CORES + lax.axis_index("c")
        pltpu.sync_copy(banned_hbm, banned_v)
        zeros16 = jnp.zeros((LANES,), jnp.float32)
        minval16 = jnp.full((LANES,), MIN_VAL, jnp.float32)
        bufs = (buf_a, buf_b)
        sems = (sem_a, sem_b)
        pending = [None, None]

        for ci in range(CHUNKS_PER_W):
            buf = bufs[ci % 2]
            sem = sems[ci % 2]
            if pending[ci % 2] is not None:
                pending[ci % 2].wait()
            base = (wid * CHUNKS_PER_W + ci) * CHUNK

            def zbody(i, c):
                for u in range(zu):
                    buf[pl.ds((i * zu + u) * LANES, LANES)] = zeros16
                return c

            lax.fori_loop(0, CHUNK // (LANES * zu), zbody, 0)

            def sbody(j, c):
                for u in range(su):
                    idx = banned_v[pl.ds((j * su + u) * LANES, LANES)]
                    local = idx - base
                    ok = (local >= 0) & (local < CHUNK)
                    safe = jnp.clip(local, 0, CHUNK - 1)
                    plsc.store_scatter(buf, [safe], minval16, mask=ok)
                return c

            lax.fori_loop(0, N_BANNED // (LANES * su), sbody, 0)
            pending[ci % 2] = pltpu.async_copy(
                buf, mask_hbm.at[pl.ds(base, CHUNK)], sem)
        for cp in pending:
            if cp is not None:
                cp.wait()

    return mask_kernel


def _tc_body(x_ref, w1_ref, b1_ref, w2_ref, b2_ref, m_ref,
             raw_ref, idx_ref, val_ref):
    h = jnp.maximum(
        jnp.dot(x_ref[...], w1_ref[...], preferred_element_type=jnp.float32)
        + b1_ref[...], 0.0)
    raw = (jnp.dot(h, w2_ref[...], preferred_element_type=jnp.float32)
           + b2_ref[...])
    raw_ref[...] = raw
    m = m_ref[...].reshape(TILE, ACTIONS)
    q = jnp.where(m < 0.0, MIN_VAL, raw)
    vmax = jnp.max(q, axis=1, keepdims=True)
    col = lax.broadcasted_iota(jnp.int32, q.shape, 1)
    idx_ref[...] = jnp.min(jnp.where(q == vmax, col, jnp.int32(ACTIONS)),
                           axis=1, keepdims=True)
    val_ref[...] = vmax


_tc_call = pl.pallas_call(
    _tc_body,
    grid=(GRID,),
    in_specs=[
        pl.BlockSpec((TILE, IN_DIM), lambda i: (i, 0)),
        pl.BlockSpec((IN_DIM, HIDDEN), lambda i: (0, 0)),
        pl.BlockSpec((1, HIDDEN), lambda i: (0, 0)),
        pl.BlockSpec((HIDDEN, ACTIONS), lambda i: (0, 0)),
        pl.BlockSpec((1, ACTIONS), lambda i: (0, 0)),
        pl.BlockSpec((TILE // 8, 8, ACTIONS // 128, 128), lambda i: (i, 0, 0, 0)),
    ],
    out_specs=[
        pl.BlockSpec((TILE, ACTIONS), lambda i: (i, 0)),
        pl.BlockSpec((TILE, 1), lambda i: (i, 0)),
        pl.BlockSpec((TILE, 1), lambda i: (i, 0)),
    ],
    out_shape=[
        jax.ShapeDtypeStruct((B, ACTIONS), jnp.float32),
        jax.ShapeDtypeStruct((B, 1), jnp.int32),
        jax.ShapeDtypeStruct((B, 1), jnp.float32),
    ],
    compiler_params=pltpu.CompilerParams(
        dimension_semantics=("arbitrary",)),
)


def kernel(encoded_graphs, banned_acts, W1, b1, W2, b2):
    mask = _make_mask_kernel()(banned_acts.astype(jnp.int32))
    mask4 = mask.reshape(B // 8, 8, ACTIONS // 128, 128)
    raw, idx, val = _tc_call(encoded_graphs, W1, b1.reshape(1, HIDDEN), W2,
                             b2.reshape(1, ACTIONS), mask4)
    return (idx, val, raw)


# speedup vs baseline: 15.6863x; 1.0994x over previous
import functools

import jax
import jax.numpy as jnp
import numpy as np
from jax import lax
from jax.experimental import pallas as pl
from jax.experimental.pallas import tpu as pltpu
from jax.experimental.pallas import tpu_sc as plsc

B = 16384
IN_DIM = 169
IN_PAD = 256
HIDDEN = 2048
ACTIONS = 1024
N_BANNED = 4096
MIN_VAL = float(np.finfo(np.float32).min)

TILE = 512
GRID = B // TILE

NUM_CORES = 2
NUM_SUBCORES = 16
NW = NUM_CORES * NUM_SUBCORES
ROWS_PER_W = B // NW
CHUNK_ROWS = 32
CHUNK = CHUNK_ROWS * ACTIONS
CHUNKS_PER_W = ROWS_PER_W // CHUNK_ROWS
LANES = 16


@functools.lru_cache(maxsize=1)
def _make_mask_kernel():
    mesh = plsc.VectorSubcoreMesh(core_axis_name="c", subcore_axis_name="s")

    zu = 16
    su = 4

    @functools.partial(
        pl.kernel,
        mesh=mesh,
        out_type=jax.ShapeDtypeStruct((B * ACTIONS,), jnp.float32),
        scratch_types=[
            pltpu.VMEM((N_BANNED,), jnp.int32),
            pltpu.VMEM((CHUNK,), jnp.float32),
            pltpu.VMEM((CHUNK,), jnp.float32),
            pltpu.SemaphoreType.DMA,
            pltpu.SemaphoreType.DMA,
        ],
        compiler_params=pltpu.CompilerParams(needs_layout_passes=False),
    )
    def mask_kernel(banned_hbm, mask_hbm, banned_v, buf_a, buf_b, sem_a, sem_b):
        wid = lax.axis_index("s") * NUM_CORES + lax.axis_index("c")
        pltpu.sync_copy(banned_hbm, banned_v)
        zeros16 = jnp.zeros((LANES,), jnp.float32)
        minval16 = jnp.full((LANES,), MIN_VAL, jnp.float32)
        bufs = (buf_a, buf_b)
        sems = (sem_a, sem_b)
        pending = [None, None]

        for ci in range(CHUNKS_PER_W):
            buf = bufs[ci % 2]
            sem = sems[ci % 2]
            if pending[ci % 2] is not None:
                pending[ci % 2].wait()
            base = (wid * CHUNKS_PER_W + ci) * CHUNK

            def zbody(i, c):
                for u in range(zu):
                    buf[pl.ds((i * zu + u) * LANES, LANES)] = zeros16
                return c

            lax.fori_loop(0, CHUNK // (LANES * zu), zbody, 0)

            def sbody(j, c):
                for u in range(su):
                    idx = banned_v[pl.ds((j * su + u) * LANES, LANES)]
                    local = idx - base
                    ok = (local >= 0) & (local < CHUNK)
                    safe = jnp.clip(local, 0, CHUNK - 1)
                    plsc.store_scatter(buf, [safe], minval16, mask=ok)
                return c

            lax.fori_loop(0, N_BANNED // (LANES * su), sbody, 0)
            pending[ci % 2] = pltpu.async_copy(
                buf, mask_hbm.at[pl.ds(base, CHUNK)], sem)
        for cp in pending:
            if cp is not None:
                cp.wait()

    return mask_kernel


def _mlp_body(x_ref, w1_ref, b1_ref, w2_ref, b2_ref, raw_ref):
    h = jnp.maximum(
        jnp.dot(x_ref[...], w1_ref[...], preferred_element_type=jnp.float32)
        + b1_ref[...], 0.0)
    raw_ref[...] = (
        jnp.dot(h, w2_ref[...], preferred_element_type=jnp.float32)
        + b2_ref[...])


_mlp_call = pl.pallas_call(
    _mlp_body,
    grid=(GRID,),
    in_specs=[
        pl.BlockSpec((TILE, IN_DIM), lambda i: (i, 0)),
        pl.BlockSpec((IN_DIM, HIDDEN), lambda i: (0, 0)),
        pl.BlockSpec((1, HIDDEN), lambda i: (0, 0)),
        pl.BlockSpec((HIDDEN, ACTIONS), lambda i: (0, 0)),
        pl.BlockSpec((1, ACTIONS), lambda i: (0, 0)),
    ],
    out_specs=pl.BlockSpec((TILE, ACTIONS), lambda i: (i, 0)),
    out_shape=jax.ShapeDtypeStruct((B, ACTIONS), jnp.float32),
    compiler_params=pltpu.CompilerParams(
        dimension_semantics=("arbitrary",)),
)

TILE2 = 2048
GRID2 = B // TILE2


def _top1_body(raw_ref, m_ref, idx_ref, val_ref):
    raw = raw_ref[...]
    m = m_ref[...].reshape(TILE2, ACTIONS)
    q = jnp.where(m < 0.0, MIN_VAL, raw)
    vmax = jnp.max(q, axis=1, keepdims=True)
    col = lax.broadcasted_iota(jnp.int32, q.shape, 1)
    idx_ref[...] = jnp.min(jnp.where(q == vmax, col, jnp.int32(ACTIONS)),
                           axis=1, keepdims=True)
    val_ref[...] = vmax


_top1_call = pl.pallas_call(
    _top1_body,
    grid=(GRID2,),
    in_specs=[
        pl.BlockSpec((TILE2, ACTIONS), lambda i: (i, 0)),
        pl.BlockSpec((TILE2 // 8, 8, ACTIONS // 128, 128),
                     lambda i: (i, 0, 0, 0)),
    ],
    out_specs=[
        pl.BlockSpec((TILE2, 1), lambda i: (i, 0)),
        pl.BlockSpec((TILE2, 1), lambda i: (i, 0)),
    ],
    out_shape=[
        jax.ShapeDtypeStruct((B, 1), jnp.int32),
        jax.ShapeDtypeStruct((B, 1), jnp.float32),
    ],
    compiler_params=pltpu.CompilerParams(
        dimension_semantics=("arbitrary",)),
)


def kernel(encoded_graphs, banned_acts, W1, b1, W2, b2):
    mask = _make_mask_kernel()(banned_acts.astype(jnp.int32))
    mask4 = mask.reshape(B // 8, 8, ACTIONS // 128, 128)
    raw = _mlp_call(encoded_graphs, W1, b1.reshape(1, HIDDEN), W2,
                    b2.reshape(1, ACTIONS))
    idx, val = _top1_call(raw, mask4)
    return (idx, val, raw)
